# baseline (speedup 1.0000x reference)
import functools

import jax
import jax.numpy as jnp
from jax import lax
from jax.experimental import pallas as pl

D = 16
V = 128
H = 256
IN_DIM = (D - 1) * V
OUT_DIM = D * V


def _mask_w1_kernel(w1_ref, o_ref):
    r = lax.broadcasted_iota(jnp.int32, (IN_DIM, H), 0)
    c = lax.broadcasted_iota(jnp.int32, (IN_DIM, H), 1)
    m = (r // V) <= (c % (D - 1))
    o_ref[...] = jnp.where(m, w1_ref[...], 0.0)


def _mask_w2_kernel(w2_ref, o_ref):
    r = lax.broadcasted_iota(jnp.int32, (H, OUT_DIM), 0)
    c = lax.broadcasted_iota(jnp.int32, (H, OUT_DIM), 1)
    m = (r % (D - 1) + 1) <= (c // V)
    o_ref[...] = jnp.where(m, w2_ref[...], 0.0)


def _made_kernel(x_ref, w1_ref, b1_ref, w2_ref, b2_ref, o_ref, *, bm):
    xb = x_ref[...]
    v_iota = lax.broadcasted_iota(jnp.int32, (bm, V), 1)
    ohs = [(xb[:, d:d + 1] == v_iota).astype(jnp.float32) for d in range(D)]
    oh_in = jnp.concatenate(ohs[:D - 1], axis=1)
    h = jnp.dot(oh_in, w1_ref[...], preferred_element_type=jnp.float32)
    h = jnp.maximum(h + b1_ref[...], 0.0)
    y = jnp.dot(h, w2_ref[...], preferred_element_type=jnp.float32)
    y = y + b2_ref[...]
    ysel = y[:, 0:V] * ohs[0]
    nprod = jnp.sum(jnp.exp(y[:, 0:V]), axis=1)
    for d in range(1, D):
        y_d = y[:, d * V:(d + 1) * V]
        ysel = ysel + y_d * ohs[d]
        nprod = nprod * jnp.sum(jnp.exp(y_d), axis=1)
    o_ref[...] = jnp.sum(ysel, axis=1) - jnp.log(nprod)


def kernel(x, W1, b1, W2, b2):
    W1m = pl.pallas_call(
        _mask_w1_kernel,
        out_shape=jax.ShapeDtypeStruct((IN_DIM, H), jnp.float32),
    )(W1)
    W2m = pl.pallas_call(
        _mask_w2_kernel,
        out_shape=jax.ShapeDtypeStruct((H, OUT_DIM), jnp.float32),
    )(W2)
    B = x.shape[0]
    bm = 2048
    out = pl.pallas_call(
        functools.partial(_made_kernel, bm=bm),
        grid=(B // bm,),
        in_specs=[
            pl.BlockSpec((bm, D), lambda i: (i, 0)),
            pl.BlockSpec((IN_DIM, H), lambda i: (0, 0)),
            pl.BlockSpec((1, H), lambda i: (0, 0)),
            pl.BlockSpec((H, OUT_DIM), lambda i: (0, 0)),
            pl.BlockSpec((1, OUT_DIM), lambda i: (0, 0)),
        ],
        out_specs=pl.BlockSpec((bm,), lambda i: (i,)),
        out_shape=jax.ShapeDtypeStruct((B,), jnp.float32),
    )(x.astype(jnp.int32), W1m, b1.reshape(1, H), W2m, b2.reshape(1, OUT_DIM))
    return out


# speedup vs baseline: 1.1577x; 1.1577x over previous
import functools

import jax
import jax.numpy as jnp
from jax import lax
from jax.experimental import pallas as pl

D = 16
V = 128
H = 256
IN_DIM = (D - 1) * V
OUT_DIM = D * V
KAUG = H + 8


def _prep_w1_kernel(w1t_ref, b1_ref, o_ref):
    r = lax.broadcasted_iota(jnp.int32, (H, IN_DIM), 0)
    c = lax.broadcasted_iota(jnp.int32, (H, IN_DIM), 1)
    m = (c // V) <= (r % (D - 1))
    w = jnp.where(m, w1t_ref[...], 0.0).astype(jnp.bfloat16)
    bias = jnp.broadcast_to(b1_ref[...], (H, V)).astype(jnp.bfloat16)
    o_ref[...] = jnp.concatenate([w, bias], axis=1)


def _prep_w2_kernel(w2t_ref, b2_ref, o_ref):
    r = lax.broadcasted_iota(jnp.int32, (OUT_DIM, H), 0)
    c = lax.broadcasted_iota(jnp.int32, (OUT_DIM, H), 1)
    m = (c % (D - 1) + 1) <= (r // V)
    w = jnp.where(m, w2t_ref[...], 0.0).astype(jnp.bfloat16)
    bias = b2_ref[...].astype(jnp.bfloat16)
    pad = jnp.zeros((OUT_DIM, KAUG - H - 1), jnp.bfloat16)
    o_ref[...] = jnp.concatenate([w, bias, pad], axis=1)


def _made_kernel(xt_ref, w1_ref, w2_ref, o_ref, *, bm):
    xt = xt_ref[...]
    v_iota = lax.broadcasted_iota(jnp.int32, (V, bm), 0)
    masks = [v_iota == jnp.broadcast_to(xt[d:d + 1, :], (V, bm))
             for d in range(D)]
    oh = jnp.concatenate([m.astype(jnp.bfloat16) for m in masks], axis=0)
    h = jnp.dot(w1_ref[...], oh, preferred_element_type=jnp.float32)
    h = jnp.maximum(h, 0.0).astype(jnp.bfloat16)
    one_row = (lax.broadcasted_iota(jnp.int32, (KAUG - H, bm), 0) == 0)
    h_aug = jnp.concatenate([h, one_row.astype(jnp.bfloat16)], axis=0)
    y = jnp.dot(w2_ref[...], h_aug, preferred_element_type=jnp.float32)
    ysel = jnp.where(masks[0], y[0:V, :], 0.0)
    nprod = jnp.sum(jnp.exp(y[0:V, :]), axis=0)
    for d in range(1, D):
        y_d = y[d * V:(d + 1) * V, :]
        ysel = ysel + jnp.where(masks[d], y_d, 0.0)
        nprod = nprod * jnp.sum(jnp.exp(y_d), axis=0)
    o_ref[...] = jnp.sum(ysel, axis=0) - jnp.log(nprod)


def kernel(x, W1, b1, W2, b2):
    W1aug = pl.pallas_call(
        _prep_w1_kernel,
        out_shape=jax.ShapeDtypeStruct((H, OUT_DIM), jnp.bfloat16),
    )(W1.T, b1.reshape(H, 1))
    W2aug = pl.pallas_call(
        _prep_w2_kernel,
        out_shape=jax.ShapeDtypeStruct((OUT_DIM, KAUG), jnp.bfloat16),
    )(W2.T, b2.reshape(OUT_DIM, 1))
    B = x.shape[0]
    bm = 1024
    xt = x.astype(jnp.int32).T
    out = pl.pallas_call(
        functools.partial(_made_kernel, bm=bm),
        grid=(B // bm,),
        in_specs=[
            pl.BlockSpec((D, bm), lambda i: (0, i)),
            pl.BlockSpec((H, OUT_DIM), lambda i: (0, 0)),
            pl.BlockSpec((OUT_DIM, KAUG), lambda i: (0, 0)),
        ],
        out_specs=pl.BlockSpec((bm,), lambda i: (i,)),
        out_shape=jax.ShapeDtypeStruct((B,), jnp.float32),
    )(xt, W1aug, W2aug)
    return out
